# Initial kernel scaffold; baseline (speedup 1.0000x reference)
import jax
import jax.numpy as jnp
from jax.experimental import pallas as pl


def kernel(x, edge_index, W1, b1, W2, b2):
    raise NotImplementedError("write your pallas kernel here")


# speedup vs baseline: 13.1843x; 13.1843x over previous
import functools

import jax
import jax.numpy as jnp
from jax import lax
from jax.experimental import pallas as pl
from jax.experimental.pallas import tpu as pltpu
from jax.experimental.pallas import tpu_sc as plsc

N = 10000
E = 320000
D = 128
DH = D // 2

NC = 2
NS = 16
K = 128
EP = 327680
EPT = EP // NS
NCHUNK = EPT // K
R = 5
G = NCHUNK // R
GD = NCHUNK // NC // R
NP = 10240
RPT = NP // NS
PAD_DST = NP - 8

_MESH = plsc.VectorSubcoreMesh(core_axis_name="c", subcore_axis_name="s")


@functools.partial(
    pl.kernel,
    out_type=jax.ShapeDtypeStruct((NC, NP, 16), jnp.float32),
    mesh=_MESH,
    scratch_types=[
        pltpu.VMEM((NCHUNK, K), jnp.int32),
        pltpu.VMEM((K, 16), jnp.float32),
        pltpu.VMEM_SHARED((NP, 16), jnp.float32),
        pltpu.SemaphoreType.DMA((R,)),
    ],
)
def _deg_kernel(edges, ones_hbm, zeros_hbm, out, didx, ones_v, acc, sem):
    c = lax.axis_index("c")
    s = lax.axis_index("s")
    pltpu.sync_copy(edges.at[1, s], didx)
    pltpu.sync_copy(ones_hbm, ones_v)
    r0 = s * RPT
    pltpu.sync_copy(zeros_hbm.at[pl.ds(r0, RPT)], acc.at[pl.ds(r0, RPT)])
    plsc.subcore_barrier()
    j0 = c * (NCHUNK // NC)

    def body(g, carry):
        for r in range(R):
            j = j0 + g * R + r
            pltpu.async_copy(ones_v, acc.at[didx.at[j]], sem.at[r], add=True)
        for r in range(R):
            j = j0 + g * R + r
            pltpu.make_async_copy(ones_v, acc.at[didx.at[j]], sem.at[r]).wait()
        return carry

    lax.fori_loop(0, GD, body, 0)
    plsc.subcore_barrier()
    pltpu.sync_copy(acc.at[pl.ds(r0, RPT)], out.at[c, pl.ds(r0, RPT)])


@functools.partial(
    pl.kernel,
    out_type=jax.ShapeDtypeStruct((NC, NP, DH), jnp.float32),
    mesh=_MESH,
    compiler_params=pltpu.CompilerParams(use_tc_tiling_on_sc=False),
    scratch_types=[
        pltpu.VMEM((NCHUNK, K), jnp.int32),
        pltpu.VMEM((NCHUNK, K), jnp.int32),
        pltpu.VMEM((R, K, DH), jnp.float32),
        pltpu.VMEM_SHARED((NP, DH), jnp.float32),
        pltpu.SemaphoreType.DMA((R,)),
        pltpu.SemaphoreType.DMA((R,)),
    ],
)
def _agg_kernel(hs, edges, zeros_hbm, out, sidx, didx, bufs, acc, gsem, ssem):
    c = lax.axis_index("c")
    s = lax.axis_index("s")
    pltpu.sync_copy(edges.at[0, s], sidx)
    pltpu.sync_copy(edges.at[1, s], didx)
    r0 = s * RPT
    pltpu.sync_copy(zeros_hbm.at[pl.ds(r0, RPT)], acc.at[pl.ds(r0, RPT)])
    plsc.subcore_barrier()

    for r in range(R):
        pltpu.async_copy(hs.at[c].at[sidx.at[r]], bufs.at[r], gsem.at[r])

    def body(g, carry):
        for r in range(R):
            j = g * R + r
            pltpu.make_async_copy(hs.at[c].at[sidx.at[j]], bufs.at[r], gsem.at[r]).wait()
            pltpu.async_copy(bufs.at[r], acc.at[didx.at[j]], ssem.at[r], add=True)

        @pl.when(g < G - 1)
        def _():
            for r in range(R):
                j = g * R + r
                pltpu.make_async_copy(bufs.at[r], acc.at[didx.at[j]], ssem.at[r]).wait()
                pltpu.async_copy(hs.at[c].at[sidx.at[j + R]], bufs.at[r], gsem.at[r])

        return carry

    lax.fori_loop(0, G, body, 0)
    for r in range(R):
        j = (G - 1) * R + r
        pltpu.make_async_copy(bufs.at[r], acc.at[didx.at[j]], ssem.at[r]).wait()
    plsc.subcore_barrier()
    pltpu.sync_copy(acc.at[pl.ds(r0, RPT)], out.at[c, pl.ds(r0, RPT)])


BM = 1000
GRID = N // BM


def _dis(degp_ref):
    deg = degp_ref[0, :, 0:1] + degp_ref[1, :, 0:1] + 1.0
    return lax.rsqrt(deg)


def _tc1_body(x_ref, w_ref, degp_ref, hs_ref):
    h = _dis(degp_ref) * jnp.dot(
        x_ref[...], w_ref[...], preferred_element_type=jnp.float32
    )
    hs_ref[0] = h[:, :DH]
    hs_ref[1] = h[:, DH:]


def _tc2_body(agg_ref, hs1_ref, degp_ref, b1_ref, w2_ref, hs2_ref):
    dis = _dis(degp_ref)
    tot = jnp.concatenate(
        [agg_ref[0] + hs1_ref[0], agg_ref[1] + hs1_ref[1]], axis=-1
    )
    h = jnp.maximum(dis * tot + b1_ref[...], 0.0)
    h2 = dis * jnp.dot(h, w2_ref[...], preferred_element_type=jnp.float32)
    hs2_ref[0] = h2[:, :DH]
    hs2_ref[1] = h2[:, DH:]


def _tc3_body(agg_ref, hs2_ref, degp_ref, b2_ref, out_ref):
    dis = _dis(degp_ref)
    tot = jnp.concatenate(
        [agg_ref[0] + hs2_ref[0], agg_ref[1] + hs2_ref[1]], axis=-1
    )
    out_ref[...] = dis * tot + b2_ref[...]


_row_spec = pl.BlockSpec((BM, D), lambda i: (i, 0))
_w_spec = pl.BlockSpec((D, D), lambda i: (0, 0))
_degp_spec = pl.BlockSpec((NC, BM, 16), lambda i: (0, i, 0))
_half_spec = pl.BlockSpec((NC, BM, DH), lambda i: (0, i, 0))
_b_spec = pl.BlockSpec((1, D), lambda i: (0, 0))
_half_out = jax.ShapeDtypeStruct((NC, N, DH), jnp.float32)

_tc1 = pl.pallas_call(
    _tc1_body,
    grid=(GRID,),
    in_specs=[_row_spec, _w_spec, _degp_spec],
    out_specs=_half_spec,
    out_shape=_half_out,
)

_tc2 = pl.pallas_call(
    _tc2_body,
    grid=(GRID,),
    in_specs=[_half_spec, _half_spec, _degp_spec, _b_spec, _w_spec],
    out_specs=_half_spec,
    out_shape=_half_out,
)

_tc3 = pl.pallas_call(
    _tc3_body,
    grid=(GRID,),
    in_specs=[_half_spec, _half_spec, _degp_spec, _b_spec],
    out_specs=_row_spec,
    out_shape=jax.ShapeDtypeStruct((N, D), jnp.float32),
)


@jax.jit
def kernel(x, edge_index, W1, b1, W2, b2):
    e32 = edge_index.astype(jnp.int32)
    pad = jnp.stack(
        [
            jnp.zeros((EP - E,), jnp.int32),
            jnp.full((EP - E,), PAD_DST, jnp.int32),
        ]
    )
    e = jnp.concatenate([e32, pad], axis=1).reshape(2, NS, NCHUNK, K)
    ones16 = jnp.ones((K, 16), jnp.float32)
    zeros16 = jnp.zeros((NP, 16), jnp.float32)
    zerosH = jnp.zeros((NP, DH), jnp.float32)

    degp = _deg_kernel(e, ones16, zeros16)[:, :N]
    hs1 = _tc1(x, W1, degp)
    agg1 = _agg_kernel(hs1, e, zerosH)[:, :N]
    hs2 = _tc2(agg1, hs1, degp, b1.reshape(1, D), W2)
    agg2 = _agg_kernel(hs2, e, zerosH)[:, :N]
    return _tc3(agg2, hs2, degp, b2.reshape(1, D))
